# baseline (speedup 1.0000x reference)
import functools

import jax
import jax.numpy as jnp
from jax import lax
from jax.experimental import pallas as pl
from jax.experimental.pallas import tpu as pltpu
from jax.experimental.pallas import tpu_sc as plsc

N_NODES = 10000
N_EDGES = 320000
D = 128

NUM_WORKERS = 32
PER_WORKER = N_EDGES // NUM_WORKERS
CHUNK = 80
N_CHUNKS = PER_WORKER // CHUNK
GROUPS = CHUNK // 16

_MESH = plsc.VectorSubcoreMesh(core_axis_name="c", subcore_axis_name="s")

_GATHER_DNUMS = lax.GatherDimensionNumbers(
    offset_dims=(), collapsed_slice_dims=(0,), start_index_map=(0,))


def _perm(v, idx):
    return lax.gather(v, idx[:, None], _GATHER_DNUMS, slice_sizes=(1,),
                      mode=lax.GatherScatterMode.PROMISE_IN_BOUNDS)


@functools.partial(
    pl.kernel,
    out_type=(
        jax.ShapeDtypeStruct((N_EDGES,), jnp.float32),
        jax.ShapeDtypeStruct((N_EDGES,), jnp.float32),
    ),
    mesh=_MESH,
    scratch_types=[
        pltpu.VMEM((PER_WORKER,), jnp.int32),
        pltpu.VMEM((PER_WORKER,), jnp.int32),
        pltpu.VMEM((PER_WORKER,), jnp.int32),
        [pltpu.VMEM((CHUNK, D), jnp.float32) for _ in range(3)],
        [pltpu.VMEM((CHUNK, D), jnp.float32) for _ in range(3)],
        [pltpu.VMEM((CHUNK, D), jnp.float32) for _ in range(3)],
        pltpu.VMEM((CHUNK,), jnp.float32),
        pltpu.VMEM((CHUNK,), jnp.float32),
        [pltpu.SemaphoreType.DMA for _ in range(3)],
    ],
)
def _sc_scores(src_hbm, tar_hbm, negi_hbm, table_hbm, pos_hbm, neg_hbm,
               sidx, tidx, nidx, srows, trows, nrows, pbuf, nbuf, sems):
    wid = lax.axis_index("s") * 2 + lax.axis_index("c")
    base_w = wid * PER_WORKER
    lane = lax.iota(jnp.int32, 16)

    pltpu.sync_copy(src_hbm.at[pl.ds(base_w, PER_WORKER)], sidx)
    pltpu.sync_copy(tar_hbm.at[pl.ds(base_w, PER_WORKER)], tidx)
    pltpu.sync_copy(negi_hbm.at[pl.ds(base_w, PER_WORKER)], nidx)

    def issue(c, slot):
        off = c * CHUNK
        pltpu.async_copy(table_hbm.at[sidx.at[pl.ds(off, CHUNK)]],
                         srows[slot], sems[slot])
        pltpu.async_copy(table_hbm.at[tidx.at[pl.ds(off, CHUNK)]],
                         trows[slot], sems[slot])
        pltpu.async_copy(table_hbm.at[nidx.at[pl.ds(off, CHUNK)]],
                         nrows[slot], sems[slot])

    def drain(c, slot):
        off = c * CHUNK
        pltpu.make_async_copy(table_hbm.at[sidx.at[pl.ds(off, CHUNK)]],
                              srows[slot], sems[slot]).wait()
        pltpu.make_async_copy(table_hbm.at[tidx.at[pl.ds(off, CHUNK)]],
                              trows[slot], sems[slot]).wait()
        pltpu.make_async_copy(table_hbm.at[nidx.at[pl.ds(off, CHUNK)]],
                              nrows[slot], sems[slot]).wait()

    def compute(c, slot):
        sr, tr, nr = srows[slot], trows[slot], nrows[slot]
        for g in range(GROUPS):

            def edge_body(k, acc):
                pvec, nvec = acc
                e = g * 16 + k
                pa = jnp.zeros((16,), jnp.float32)
                na = jnp.zeros((16,), jnp.float32)
                for j in range(D // 16):
                    s = sr[e, pl.ds(16 * j, 16)]
                    t = tr[e, pl.ds(16 * j, 16)]
                    n = nr[e, pl.ds(16 * j, 16)]
                    pa = pa + s * t
                    na = na + s * n
                for sh in (8, 4, 2, 1):
                    perm = lane ^ sh
                    pa = pa + _perm(pa, perm)
                    na = na + _perm(na, perm)
                sel = lane == k
                pvec = jnp.where(sel, pa, pvec)
                nvec = jnp.where(sel, na, nvec)
                return pvec, nvec

            zero = jnp.zeros((16,), jnp.float32)
            pvec, nvec = lax.fori_loop(0, 16, edge_body, (zero, zero))
            pbuf[pl.ds(g * 16, 16)] = pvec
            nbuf[pl.ds(g * 16, 16)] = nvec
        base = base_w + c * CHUNK
        pltpu.sync_copy(pbuf, pos_hbm.at[pl.ds(base, CHUNK)])
        pltpu.sync_copy(nbuf, neg_hbm.at[pl.ds(base, CHUNK)])

    issue(0, 0)
    issue(1, 1)

    def chunk_triple(c3, carry):
        for b in range(3):
            c = 3 * c3 + b
            issue(c + 2, (b + 2) % 3)
            drain(c, b)
            compute(c, b)
        return carry

    lax.fori_loop(0, (N_CHUNKS - 2) // 3, chunk_triple, 0)
    drain(N_CHUNKS - 2, 0)
    compute(N_CHUNKS - 2, 0)
    drain(N_CHUNKS - 1, 1)
    compute(N_CHUNKS - 1, 1)


def _tc_finalize(pos_ref, neg_ref, table_ref, out_ref):
    pos = pos_ref[...]
    neg = neg_ref[...]
    pos_loss = jnp.mean(jax.nn.softplus(-pos))
    neg_loss = jnp.mean(jax.nn.softplus(neg))
    col_mean = jnp.mean(table_ref[...], axis=0)
    avg_loss = jnp.mean(jnp.log(col_mean + 0.0001))
    out_ref[0, 0] = pos_loss + neg_loss - avg_loss


def kernel(edges, cluster_logits):
    neg_idx = jax.random.randint(
        jax.random.key(42), (edges.shape[1],), 0, cluster_logits.shape[0],
        dtype=jnp.int32)
    src_ids = edges[0]
    tar_ids = edges[1]
    pos_score, neg_score = _sc_scores(src_ids, tar_ids, neg_idx,
                                      cluster_logits)
    out = pl.pallas_call(
        _tc_finalize,
        out_shape=jax.ShapeDtypeStruct((1, 1), jnp.float32),
        out_specs=pl.BlockSpec(memory_space=pltpu.SMEM),
    )(pos_score, neg_score, cluster_logits)
    return out[0, 0]


# speedup vs baseline: 1.0015x; 1.0015x over previous
import functools

import jax
import jax.numpy as jnp
from jax import lax
from jax.experimental import pallas as pl
from jax.experimental.pallas import tpu as pltpu
from jax.experimental.pallas import tpu_sc as plsc

N_NODES = 10000
N_EDGES = 320000
D = 128

NUM_WORKERS = 32
PER_WORKER = N_EDGES // NUM_WORKERS
CHUNK = 80
N_CHUNKS = PER_WORKER // CHUNK
GROUPS = CHUNK // 16

_MESH = plsc.VectorSubcoreMesh(core_axis_name="c", subcore_axis_name="s")

_GATHER_DNUMS = lax.GatherDimensionNumbers(
    offset_dims=(), collapsed_slice_dims=(0,), start_index_map=(0,))


def _perm(v, idx):
    return lax.gather(v, idx[:, None], _GATHER_DNUMS, slice_sizes=(1,),
                      mode=lax.GatherScatterMode.PROMISE_IN_BOUNDS)


@functools.partial(
    pl.kernel,
    out_type=(
        jax.ShapeDtypeStruct((N_EDGES,), jnp.float32),
        jax.ShapeDtypeStruct((N_EDGES,), jnp.float32),
    ),
    mesh=_MESH,
    scratch_types=[
        pltpu.VMEM((PER_WORKER,), jnp.int32),
        pltpu.VMEM((PER_WORKER,), jnp.int32),
        pltpu.VMEM((PER_WORKER,), jnp.int32),
        [pltpu.VMEM((CHUNK, D), jnp.float32) for _ in range(3)],
        [pltpu.VMEM((CHUNK, D), jnp.float32) for _ in range(3)],
        [pltpu.VMEM((CHUNK, D), jnp.float32) for _ in range(3)],
        pltpu.VMEM((CHUNK,), jnp.float32),
        pltpu.VMEM((CHUNK,), jnp.float32),
        [pltpu.SemaphoreType.DMA for _ in range(3)],
    ],
)
def _sc_scores(src_hbm, tar_hbm, negi_hbm, table_hbm, pos_hbm, neg_hbm,
               sidx, tidx, nidx, srows, trows, nrows, pbuf, nbuf, sems):
    wid = lax.axis_index("s") * 2 + lax.axis_index("c")
    base_w = wid * PER_WORKER
    lane = lax.iota(jnp.int32, 16)

    pltpu.sync_copy(src_hbm.at[pl.ds(base_w, PER_WORKER)], sidx)
    pltpu.sync_copy(tar_hbm.at[pl.ds(base_w, PER_WORKER)], tidx)
    pltpu.sync_copy(negi_hbm.at[pl.ds(base_w, PER_WORKER)], nidx)

    H = CHUNK // 2

    def issue(c, slot):
        off = c * CHUNK
        for idx, rows in ((sidx, srows), (tidx, trows), (nidx, nrows)):
            pltpu.async_copy(table_hbm.at[idx.at[pl.ds(off, H)]],
                             rows[slot].at[pl.ds(0, H)], sems[slot])
            pltpu.async_copy(table_hbm.at[idx.at[pl.ds(off + H, H)]],
                             rows[slot].at[pl.ds(H, H)], sems[slot])

    def drain(c, slot):
        off = c * CHUNK
        for idx, rows in ((sidx, srows), (tidx, trows), (nidx, nrows)):
            pltpu.make_async_copy(table_hbm.at[idx.at[pl.ds(off, H)]],
                                  rows[slot].at[pl.ds(0, H)],
                                  sems[slot]).wait()
            pltpu.make_async_copy(table_hbm.at[idx.at[pl.ds(off + H, H)]],
                                  rows[slot].at[pl.ds(H, H)],
                                  sems[slot]).wait()

    def compute(c, slot):
        sr, tr, nr = srows[slot], trows[slot], nrows[slot]
        for g in range(GROUPS):

            def edge_body(k, acc):
                pvec, nvec = acc
                e = g * 16 + k
                pa = jnp.zeros((16,), jnp.float32)
                na = jnp.zeros((16,), jnp.float32)
                for j in range(D // 16):
                    s = sr[e, pl.ds(16 * j, 16)]
                    t = tr[e, pl.ds(16 * j, 16)]
                    n = nr[e, pl.ds(16 * j, 16)]
                    pa = pa + s * t
                    na = na + s * n
                for sh in (8, 4, 2, 1):
                    perm = lane ^ sh
                    pa = pa + _perm(pa, perm)
                    na = na + _perm(na, perm)
                sel = lane == k
                pvec = jnp.where(sel, pa, pvec)
                nvec = jnp.where(sel, na, nvec)
                return pvec, nvec

            zero = jnp.zeros((16,), jnp.float32)
            pvec, nvec = lax.fori_loop(0, 16, edge_body, (zero, zero))
            pbuf[pl.ds(g * 16, 16)] = pvec
            nbuf[pl.ds(g * 16, 16)] = nvec
        base = base_w + c * CHUNK
        pltpu.sync_copy(pbuf, pos_hbm.at[pl.ds(base, CHUNK)])
        pltpu.sync_copy(nbuf, neg_hbm.at[pl.ds(base, CHUNK)])

    issue(0, 0)
    issue(1, 1)

    def chunk_triple(c3, carry):
        for b in range(3):
            c = 3 * c3 + b
            issue(c + 2, (b + 2) % 3)
            drain(c, b)
            compute(c, b)
        return carry

    lax.fori_loop(0, (N_CHUNKS - 2) // 3, chunk_triple, 0)
    drain(N_CHUNKS - 2, 0)
    compute(N_CHUNKS - 2, 0)
    drain(N_CHUNKS - 1, 1)
    compute(N_CHUNKS - 1, 1)


def _tc_finalize(pos_ref, neg_ref, table_ref, out_ref):
    pos = pos_ref[...]
    neg = neg_ref[...]
    pos_loss = jnp.mean(jax.nn.softplus(-pos))
    neg_loss = jnp.mean(jax.nn.softplus(neg))
    col_mean = jnp.mean(table_ref[...], axis=0)
    avg_loss = jnp.mean(jnp.log(col_mean + 0.0001))
    out_ref[0, 0] = pos_loss + neg_loss - avg_loss


def kernel(edges, cluster_logits):
    neg_idx = jax.random.randint(
        jax.random.key(42), (edges.shape[1],), 0, cluster_logits.shape[0],
        dtype=jnp.int32)
    src_ids = edges[0]
    tar_ids = edges[1]
    pos_score, neg_score = _sc_scores(src_ids, tar_ids, neg_idx,
                                      cluster_logits)
    out = pl.pallas_call(
        _tc_finalize,
        out_shape=jax.ShapeDtypeStruct((1, 1), jnp.float32),
        out_specs=pl.BlockSpec(memory_space=pltpu.SMEM),
    )(pos_score.reshape(2500, D), neg_score.reshape(2500, D),
      cluster_logits)
    return out[0, 0]


# speedup vs baseline: 1.0075x; 1.0059x over previous
import functools

import jax
import jax.numpy as jnp
from jax import lax
from jax.experimental import pallas as pl
from jax.experimental.pallas import tpu as pltpu
from jax.experimental.pallas import tpu_sc as plsc

N_NODES = 10000
N_EDGES = 320000
D = 128

NUM_WORKERS = 32
PER_WORKER = N_EDGES // NUM_WORKERS
CHUNK = 80
N_CHUNKS = PER_WORKER // CHUNK
GROUPS = CHUNK // 16

_MESH = plsc.VectorSubcoreMesh(core_axis_name="c", subcore_axis_name="s")

_GATHER_DNUMS = lax.GatherDimensionNumbers(
    offset_dims=(), collapsed_slice_dims=(0,), start_index_map=(0,))


def _perm(v, idx):
    return lax.gather(v, idx[:, None], _GATHER_DNUMS, slice_sizes=(1,),
                      mode=lax.GatherScatterMode.PROMISE_IN_BOUNDS)


@functools.partial(
    pl.kernel,
    out_type=(
        jax.ShapeDtypeStruct((N_EDGES,), jnp.float32),
        jax.ShapeDtypeStruct((N_EDGES,), jnp.float32),
    ),
    mesh=_MESH,
    scratch_types=[
        pltpu.VMEM((PER_WORKER,), jnp.int32),
        pltpu.VMEM((PER_WORKER,), jnp.int32),
        pltpu.VMEM((PER_WORKER,), jnp.int32),
        [pltpu.VMEM((CHUNK, D), jnp.float32) for _ in range(3)],
        [pltpu.VMEM((CHUNK, D), jnp.float32) for _ in range(3)],
        [pltpu.VMEM((CHUNK, D), jnp.float32) for _ in range(3)],
        pltpu.VMEM((CHUNK,), jnp.float32),
        pltpu.VMEM((CHUNK,), jnp.float32),
        [pltpu.SemaphoreType.DMA for _ in range(3)],
    ],
)
def _sc_scores(src_hbm, tar_hbm, negi_hbm, table_hbm, pos_hbm, neg_hbm,
               sidx, tidx, nidx, srows, trows, nrows, pbuf, nbuf, sems):
    wid = lax.axis_index("s") * 2 + lax.axis_index("c")
    base_w = wid * PER_WORKER
    lane = lax.iota(jnp.int32, 16)

    pltpu.sync_copy(src_hbm.at[pl.ds(base_w, PER_WORKER)], sidx)
    pltpu.sync_copy(tar_hbm.at[pl.ds(base_w, PER_WORKER)], tidx)
    pltpu.sync_copy(negi_hbm.at[pl.ds(base_w, PER_WORKER)], nidx)

    def issue(c, slot):
        off = c * CHUNK
        for idx, rows in ((sidx, srows), (tidx, trows), (nidx, nrows)):
            pltpu.async_copy(table_hbm.at[idx.at[pl.ds(off, CHUNK)]],
                             rows[slot], sems[slot])

    def drain(c, slot):
        off = c * CHUNK
        for idx, rows in ((sidx, srows), (tidx, trows), (nidx, nrows)):
            pltpu.make_async_copy(table_hbm.at[idx.at[pl.ds(off, CHUNK)]],
                                  rows[slot], sems[slot]).wait()

    def compute(c, slot):
        sr, tr, nr = srows[slot], trows[slot], nrows[slot]
        for g in range(GROUPS):

            def edge_body(k, acc):
                pvec, nvec = acc
                e = g * 16 + k
                pa = jnp.zeros((16,), jnp.float32)
                na = jnp.zeros((16,), jnp.float32)
                for j in range(D // 16):
                    s = sr[e, pl.ds(16 * j, 16)]
                    t = tr[e, pl.ds(16 * j, 16)]
                    n = nr[e, pl.ds(16 * j, 16)]
                    pa = pa + s * t
                    na = na + s * n
                for sh in (8, 4, 2, 1):
                    perm = lane ^ sh
                    pa = pa + _perm(pa, perm)
                    na = na + _perm(na, perm)
                sel = lane == k
                pvec = jnp.where(sel, pa, pvec)
                nvec = jnp.where(sel, na, nvec)
                return pvec, nvec

            zero = jnp.zeros((16,), jnp.float32)
            pvec, nvec = lax.fori_loop(0, 16, edge_body, (zero, zero))
            pbuf[pl.ds(g * 16, 16)] = pvec
            nbuf[pl.ds(g * 16, 16)] = nvec
        base = base_w + c * CHUNK
        pltpu.sync_copy(pbuf, pos_hbm.at[pl.ds(base, CHUNK)])
        pltpu.sync_copy(nbuf, neg_hbm.at[pl.ds(base, CHUNK)])

    issue(0, 0)
    issue(1, 1)

    def chunk_triple(c3, carry):
        for b in range(3):
            c = 3 * c3 + b
            issue(c + 2, (b + 2) % 3)
            drain(c, b)
            compute(c, b)
        return carry

    lax.fori_loop(0, (N_CHUNKS - 2) // 3, chunk_triple, 0)
    drain(N_CHUNKS - 2, 0)
    compute(N_CHUNKS - 2, 0)
    drain(N_CHUNKS - 1, 1)
    compute(N_CHUNKS - 1, 1)


def _tc_finalize(pos_ref, neg_ref, table_ref, out_ref):
    pos = pos_ref[...]
    neg = neg_ref[...]
    pos_loss = jnp.mean(jax.nn.softplus(-pos))
    neg_loss = jnp.mean(jax.nn.softplus(neg))
    col_mean = jnp.mean(table_ref[...], axis=0)
    avg_loss = jnp.mean(jnp.log(col_mean + 0.0001))
    out_ref[0, 0] = pos_loss + neg_loss - avg_loss


def kernel(edges, cluster_logits):
    neg_idx = jax.random.randint(
        jax.random.key(42), (edges.shape[1],), 0, cluster_logits.shape[0],
        dtype=jnp.int32)
    src_ids = edges[0]
    tar_ids = edges[1]
    pos_score, neg_score = _sc_scores(src_ids, tar_ids, neg_idx,
                                      cluster_logits)
    out = pl.pallas_call(
        _tc_finalize,
        out_shape=jax.ShapeDtypeStruct((1, 1), jnp.float32),
        out_specs=pl.BlockSpec(memory_space=pltpu.SMEM),
    )(pos_score.reshape(2500, D), neg_score.reshape(2500, D),
      cluster_logits)
    return out[0, 0]
